# baseline (speedup 1.0000x reference)
import functools

import jax
import jax.numpy as jnp
from jax import lax
from jax.experimental import pallas as pl
from jax.experimental.pallas import tpu as pltpu
from jax.experimental.pallas import tpu_sc as plsc

NUM_GROUP_K = 512
GROUP_SIZE_K = 32
ROW_PAD = 16
GBLK = 256


def _fps_body(xyzc_ref, cx_ref, cy_ref, cz_ref, dist_ref):
    B3, N = xyzc_ref.shape
    B = B3 // 3
    G = cx_ref.shape[1]
    xyzc = xyzc_ref[...]
    x = xyzc[0:B]
    y = xyzc[B:2 * B]
    z = xyzc[2 * B:3 * B]
    flane = lax.broadcasted_iota(jnp.int32, (B, N), 1).astype(jnp.float32)
    gcol = lax.broadcasted_iota(jnp.int32, (B, G), 1)
    bigf = jnp.float32(2.0 * N)

    dist_ref[...] = jnp.full((B, N), jnp.inf, dtype=jnp.float32)
    lx0 = x[:, 0:1]
    ly0 = y[:, 0:1]
    lz0 = z[:, 0:1]
    cx0 = jnp.where(gcol == 0, lx0, 0.0)
    cy0 = jnp.where(gcol == 0, ly0, 0.0)
    cz0 = jnp.where(gcol == 0, lz0, 0.0)

    def step(j, carry):
        lx, ly, lz, cx, cy, cz = carry
        dx = x - lx
        dy = y - ly
        dz = z - lz
        d = (dx * dx + dy * dy) + dz * dz
        dist = jnp.minimum(dist_ref[...], d)
        dist_ref[...] = dist
        mx = jnp.max(dist, axis=1, keepdims=True)
        nxt = jnp.min(jnp.where(dist == mx, flane, bigf), axis=1, keepdims=True)
        sel = flane == nxt
        sel3 = jnp.concatenate([sel, sel, sel], axis=0)
        l = jnp.sum(jnp.where(sel3, xyzc, 0.0), axis=1, keepdims=True)
        lx = l[0:B]
        ly = l[B:2 * B]
        lz = l[2 * B:3 * B]
        hit = gcol == j
        cx = cx + jnp.where(hit, lx, 0.0)
        cy = cy + jnp.where(hit, ly, 0.0)
        cz = cz + jnp.where(hit, lz, 0.0)
        return lx, ly, lz, cx, cy, cz

    _, _, _, cx, cy, cz = lax.fori_loop(
        1, G, step, (lx0, ly0, lz0, cx0, cy0, cz0))
    cx_ref[...] = cx
    cy_ref[...] = cy
    cz_ref[...] = cz


def _fps_centers(x, y, z):
    B, N = x.shape
    G = NUM_GROUP_K
    out = jax.ShapeDtypeStruct((B, G), jnp.float32)
    xyzc = jnp.concatenate([x, y, z], axis=0)
    return pl.pallas_call(
        _fps_body,
        out_shape=(out, out, out),
        scratch_shapes=[pltpu.VMEM((B, N), jnp.float32)],
    )(xyzc)


def _knn_body(x_ref, y_ref, z_ref, c_ref, idx_ref, d_ref):
    N = x_ref.shape[2]
    M = idx_ref.shape[2]
    x = x_ref[0]
    y = y_ref[0]
    z = z_ref[0]
    c = c_ref[0]
    dx = c[:, 0:1] - x
    dy = c[:, 1:2] - y
    dz = c[:, 2:3] - z
    d0 = (dx * dx + dy * dy) + dz * dz
    d_ref[...] = d0
    flane = lax.broadcasted_iota(jnp.int32, (GBLK, N), 1).astype(jnp.float32)
    mcol = lax.broadcasted_iota(jnp.int32, (GBLK, M), 1)
    bigf = jnp.float32(2.0 * N)
    mn0 = jnp.min(d0, axis=1, keepdims=True)

    def body(j, carry):
        acc, mn = carry
        dcur = d_ref[...]
        am = jnp.min(jnp.where(dcur == mn, flane, bigf), axis=1, keepdims=True)
        dnew = jnp.where(flane == am, jnp.inf, dcur)
        d_ref[...] = dnew
        mn2 = jnp.min(dnew, axis=1, keepdims=True)
        return acc + jnp.where(mcol == j, am.astype(jnp.int32), 0), mn2

    acc, _ = lax.fori_loop(
        0, M, body, (jnp.zeros((GBLK, M), jnp.int32), mn0))
    idx_ref[0] = acc


def _knn_topk(x, y, z, center):
    B, N = x.shape
    G = NUM_GROUP_K
    M = GROUP_SIZE_K
    grid = (B, G // GBLK)
    x3 = x[:, None, :]
    y3 = y[:, None, :]
    z3 = z[:, None, :]
    return pl.pallas_call(
        _knn_body,
        grid=grid,
        in_specs=[
            pl.BlockSpec((1, 1, N), lambda b, g: (b, 0, 0)),
            pl.BlockSpec((1, 1, N), lambda b, g: (b, 0, 0)),
            pl.BlockSpec((1, 1, N), lambda b, g: (b, 0, 0)),
            pl.BlockSpec((1, GBLK, 3), lambda b, g: (b, g, 0)),
        ],
        out_specs=pl.BlockSpec((1, GBLK, M), lambda b, g: (b, g, 0)),
        out_shape=jax.ShapeDtypeStruct((B, G, M), jnp.int32),
        scratch_shapes=[pltpu.VMEM((GBLK, N), jnp.float32)],
    )(x3, y3, z3, center)


def _sc_gather_normalize(flat_idx, pts_pad, cent_pad):
    R = flat_idx.shape[0]
    info = plsc.get_sparse_core_info()
    nw = info.num_cores * info.num_subcores
    rpw = R // nw
    gpw = rpw // GROUP_SIZE_K
    mesh = plsc.VectorSubcoreMesh(core_axis_name="c", subcore_axis_name="s")

    @functools.partial(
        pl.kernel,
        mesh=mesh,
        compiler_params=pltpu.CompilerParams(use_tc_tiling_on_sc=False),
        out_type=jax.ShapeDtypeStruct((R, ROW_PAD), jnp.float32),
        scratch_types=[
            pltpu.VMEM((rpw,), jnp.int32),
            pltpu.VMEM((rpw, ROW_PAD), jnp.float32),
            pltpu.VMEM((gpw, ROW_PAD), jnp.float32),
            pltpu.SemaphoreType.DMA,
        ],
    )
    def k(idx_hbm, pts_hbm, cent_hbm, out_hbm, idx_v, rows_v, cent_v, sem):
        wid = lax.axis_index("s") * info.num_cores + lax.axis_index("c")
        rbase = wid * rpw
        pltpu.sync_copy(idx_hbm.at[pl.ds(rbase, rpw)], idx_v)
        pltpu.async_copy(pts_hbm.at[idx_v], rows_v, sem).wait()
        pltpu.sync_copy(cent_hbm.at[pl.ds(wid * gpw, gpw)], cent_v)

        def body(g, _):
            cvec = cent_v[g]
            base = g * GROUP_SIZE_K
            for j in range(GROUP_SIZE_K):
                rows_v[base + j] = rows_v[base + j] - cvec
            return 0

        lax.fori_loop(0, gpw, body, 0)
        pltpu.sync_copy(rows_v, out_hbm.at[pl.ds(rbase, rpw)])

    return k(flat_idx, pts_pad, cent_pad)


def kernel(pts):
    B, N, C = pts.shape
    G = NUM_GROUP_K
    M = GROUP_SIZE_K
    x = pts[:, :, 0]
    y = pts[:, :, 1]
    z = pts[:, :, 2]
    cx, cy, cz = _fps_centers(x, y, z)
    center = jnp.stack([cx, cy, cz], axis=-1)
    idx = _knn_topk(x, y, z, center)
    flat_idx = (idx + jnp.arange(B, dtype=jnp.int32)[:, None, None] * N
                ).reshape(B * G * M)
    pts_pad = jnp.pad(pts.reshape(B * N, C), ((0, 0), (0, ROW_PAD - C)))
    cent_pad = jnp.pad(center.reshape(B * G, 3), ((0, 0), (0, ROW_PAD - 3)))
    rows = _sc_gather_normalize(flat_idx, pts_pad, cent_pad)
    neighborhood = rows[:, :C].reshape(B, G, M, C)
    return neighborhood, center


# speedup vs baseline: 1.0603x; 1.0603x over previous
import functools

import jax
import jax.numpy as jnp
from jax import lax
from jax.experimental import pallas as pl
from jax.experimental.pallas import tpu as pltpu
from jax.experimental.pallas import tpu_sc as plsc

NUM_GROUP_K = 512
GROUP_SIZE_K = 32
ROW_PAD = 16
GBLK = 256


def _fps_body(x_ref, y_ref, z_ref, cx_ref, cy_ref, cz_ref, dist_ref):
    B, N = x_ref.shape
    G = cx_ref.shape[1]
    x = x_ref[...]
    y = y_ref[...]
    z = z_ref[...]
    flane = lax.broadcasted_iota(jnp.int32, (B, N), 1).astype(jnp.float32)
    gcol = lax.broadcasted_iota(jnp.int32, (B, G), 1)
    bigf = jnp.float32(2.0 * N)

    dist_ref[...] = jnp.full((B, N), jnp.inf, dtype=jnp.float32)
    lx0 = x[:, 0:1]
    ly0 = y[:, 0:1]
    lz0 = z[:, 0:1]
    cx0 = jnp.where(gcol == 0, lx0, 0.0)
    cy0 = jnp.where(gcol == 0, ly0, 0.0)
    cz0 = jnp.where(gcol == 0, lz0, 0.0)

    def step(j, carry):
        lx, ly, lz, cx, cy, cz = carry
        dx = x - lx
        dy = y - ly
        dz = z - lz
        d = (dx * dx + dy * dy) + dz * dz
        dist = jnp.minimum(dist_ref[...], d)
        dist_ref[...] = dist
        mx = jnp.max(dist, axis=1, keepdims=True)
        nxt = jnp.min(jnp.where(dist == mx, flane, bigf), axis=1, keepdims=True)
        sel = flane == nxt
        lx = jnp.sum(jnp.where(sel, x, 0.0), axis=1, keepdims=True)
        ly = jnp.sum(jnp.where(sel, y, 0.0), axis=1, keepdims=True)
        lz = jnp.sum(jnp.where(sel, z, 0.0), axis=1, keepdims=True)
        hit = gcol == j
        cx = cx + jnp.where(hit, lx, 0.0)
        cy = cy + jnp.where(hit, ly, 0.0)
        cz = cz + jnp.where(hit, lz, 0.0)
        return lx, ly, lz, cx, cy, cz

    _, _, _, cx, cy, cz = lax.fori_loop(
        1, G, step, (lx0, ly0, lz0, cx0, cy0, cz0))
    cx_ref[...] = cx
    cy_ref[...] = cy
    cz_ref[...] = cz


def _fps_centers(x, y, z):
    B, N = x.shape
    G = NUM_GROUP_K
    out = jax.ShapeDtypeStruct((B, G), jnp.float32)
    return pl.pallas_call(
        _fps_body,
        out_shape=(out, out, out),
        scratch_shapes=[pltpu.VMEM((B, N), jnp.float32)],
    )(x, y, z)


def _knn_body(x_ref, y_ref, z_ref, c_ref, idx_ref, d_ref):
    N = x_ref.shape[2]
    M = idx_ref.shape[2]
    x = x_ref[0]
    y = y_ref[0]
    z = z_ref[0]
    c = c_ref[0]
    dx = c[:, 0:1] - x
    dy = c[:, 1:2] - y
    dz = c[:, 2:3] - z
    d0 = (dx * dx + dy * dy) + dz * dz
    d_ref[...] = d0
    flane = lax.broadcasted_iota(jnp.int32, (GBLK, N), 1).astype(jnp.float32)
    mcol = lax.broadcasted_iota(jnp.int32, (GBLK, M), 1)
    bigf = jnp.float32(2.0 * N)
    mn0 = jnp.min(d0, axis=1, keepdims=True)

    def body(j2, carry):
        acc, mn = carry
        dcur = d_ref[...]
        am1 = jnp.min(jnp.where(dcur == mn, flane, bigf), axis=1,
                      keepdims=True)
        d1 = jnp.where(flane == am1, jnp.inf, dcur)
        mn1 = jnp.min(d1, axis=1, keepdims=True)
        am2 = jnp.min(jnp.where(d1 == mn1, flane, bigf), axis=1,
                      keepdims=True)
        d2 = jnp.where(flane == am2, jnp.inf, d1)
        d_ref[...] = d2
        mn2 = jnp.min(d2, axis=1, keepdims=True)
        acc = (acc + jnp.where(mcol == 2 * j2, am1.astype(jnp.int32), 0)
               + jnp.where(mcol == 2 * j2 + 1, am2.astype(jnp.int32), 0))
        return acc, mn2

    acc, _ = lax.fori_loop(
        0, M // 2, body, (jnp.zeros((GBLK, M), jnp.int32), mn0))
    idx_ref[0] = acc


def _knn_topk(x, y, z, center):
    B, N = x.shape
    G = NUM_GROUP_K
    M = GROUP_SIZE_K
    grid = (B, G // GBLK)
    x3 = x[:, None, :]
    y3 = y[:, None, :]
    z3 = z[:, None, :]
    return pl.pallas_call(
        _knn_body,
        grid=grid,
        in_specs=[
            pl.BlockSpec((1, 1, N), lambda b, g: (b, 0, 0)),
            pl.BlockSpec((1, 1, N), lambda b, g: (b, 0, 0)),
            pl.BlockSpec((1, 1, N), lambda b, g: (b, 0, 0)),
            pl.BlockSpec((1, GBLK, 3), lambda b, g: (b, g, 0)),
        ],
        out_specs=pl.BlockSpec((1, GBLK, M), lambda b, g: (b, g, 0)),
        out_shape=jax.ShapeDtypeStruct((B, G, M), jnp.int32),
        scratch_shapes=[pltpu.VMEM((GBLK, N), jnp.float32)],
    )(x3, y3, z3, center)


def _sc_gather_normalize(flat_idx, pts_pad, cent_pad):
    R = flat_idx.shape[0]
    info = plsc.get_sparse_core_info()
    nw = info.num_cores * info.num_subcores
    rpw = R // nw
    gpw = rpw // GROUP_SIZE_K
    mesh = plsc.VectorSubcoreMesh(core_axis_name="c", subcore_axis_name="s")

    @functools.partial(
        pl.kernel,
        mesh=mesh,
        compiler_params=pltpu.CompilerParams(use_tc_tiling_on_sc=False),
        out_type=jax.ShapeDtypeStruct((R, ROW_PAD), jnp.float32),
        scratch_types=[
            pltpu.VMEM((rpw,), jnp.int32),
            pltpu.VMEM((rpw, ROW_PAD), jnp.float32),
            pltpu.VMEM((gpw, ROW_PAD), jnp.float32),
            pltpu.SemaphoreType.DMA,
        ],
    )
    def k(idx_hbm, pts_hbm, cent_hbm, out_hbm, idx_v, rows_v, cent_v, sem):
        wid = lax.axis_index("s") * info.num_cores + lax.axis_index("c")
        rbase = wid * rpw
        pltpu.sync_copy(idx_hbm.at[pl.ds(rbase, rpw)], idx_v)
        pltpu.async_copy(pts_hbm.at[idx_v], rows_v, sem).wait()
        pltpu.sync_copy(cent_hbm.at[pl.ds(wid * gpw, gpw)], cent_v)

        def body(g, _):
            cvec = cent_v[g]
            base = g * GROUP_SIZE_K
            for j in range(GROUP_SIZE_K):
                rows_v[base + j] = rows_v[base + j] - cvec
            return 0

        lax.fori_loop(0, gpw, body, 0)
        pltpu.sync_copy(rows_v, out_hbm.at[pl.ds(rbase, rpw)])

    return k(flat_idx, pts_pad, cent_pad)


def kernel(pts):
    B, N, C = pts.shape
    G = NUM_GROUP_K
    M = GROUP_SIZE_K
    x = pts[:, :, 0]
    y = pts[:, :, 1]
    z = pts[:, :, 2]
    cx, cy, cz = _fps_centers(x, y, z)
    center = jnp.stack([cx, cy, cz], axis=-1)
    idx = _knn_topk(x, y, z, center)
    flat_idx = (idx + jnp.arange(B, dtype=jnp.int32)[:, None, None] * N
                ).reshape(B * G * M)
    pts_pad = jnp.pad(pts.reshape(B * N, C), ((0, 0), (0, ROW_PAD - C)))
    cent_pad = jnp.pad(center.reshape(B * G, 3), ((0, 0), (0, ROW_PAD - 3)))
    rows = _sc_gather_normalize(flat_idx, pts_pad, cent_pad)
    neighborhood = rows[:, :C].reshape(B, G, M, C)
    return neighborhood, center


# speedup vs baseline: 1.0986x; 1.0361x over previous
import functools

import jax
import jax.numpy as jnp
from jax import lax
from jax.experimental import pallas as pl
from jax.experimental.pallas import tpu as pltpu
from jax.experimental.pallas import tpu_sc as plsc

NUM_GROUP_K = 512
GROUP_SIZE_K = 32
ROW_PAD = 16
GBLK = 256


def _fps_body(x_ref, y_ref, z_ref, cx_ref, cy_ref, cz_ref, dist_ref):
    B, N = x_ref.shape
    G = cx_ref.shape[1]
    x = x_ref[...]
    y = y_ref[...]
    z = z_ref[...]
    flane = lax.broadcasted_iota(jnp.int32, (B, N), 1).astype(jnp.float32)
    gcol = lax.broadcasted_iota(jnp.int32, (B, G), 1)
    bigf = jnp.float32(2.0 * N)

    dist_ref[...] = jnp.full((B, N), jnp.inf, dtype=jnp.float32)
    lx0 = x[:, 0:1]
    ly0 = y[:, 0:1]
    lz0 = z[:, 0:1]
    cx0 = jnp.where(gcol == 0, lx0, 0.0)
    cy0 = jnp.where(gcol == 0, ly0, 0.0)
    cz0 = jnp.where(gcol == 0, lz0, 0.0)

    def step(j, carry):
        lx, ly, lz, cx, cy, cz = carry
        dx = x - lx
        dy = y - ly
        dz = z - lz
        d = (dx * dx + dy * dy) + dz * dz
        dist = jnp.minimum(dist_ref[...], d)
        dist_ref[...] = dist
        mx = jnp.max(dist, axis=1, keepdims=True)
        nxt = jnp.min(jnp.where(dist == mx, flane, bigf), axis=1, keepdims=True)
        sel = flane == nxt
        lx = jnp.sum(jnp.where(sel, x, 0.0), axis=1, keepdims=True)
        ly = jnp.sum(jnp.where(sel, y, 0.0), axis=1, keepdims=True)
        lz = jnp.sum(jnp.where(sel, z, 0.0), axis=1, keepdims=True)
        hit = gcol == j
        cx = cx + jnp.where(hit, lx, 0.0)
        cy = cy + jnp.where(hit, ly, 0.0)
        cz = cz + jnp.where(hit, lz, 0.0)
        return lx, ly, lz, cx, cy, cz

    _, _, _, cx, cy, cz = lax.fori_loop(
        1, G, step, (lx0, ly0, lz0, cx0, cy0, cz0))
    cx_ref[...] = cx
    cy_ref[...] = cy
    cz_ref[...] = cz


def _fps_centers(x, y, z):
    B, N = x.shape
    G = NUM_GROUP_K
    out = jax.ShapeDtypeStruct((B, G), jnp.float32)
    return pl.pallas_call(
        _fps_body,
        out_shape=(out, out, out),
        scratch_shapes=[pltpu.VMEM((B, N), jnp.float32)],
    )(x, y, z)


def _knn_body(x_ref, y_ref, z_ref, c_ref, idx_ref, d_ref):
    N = x_ref.shape[2]
    M = idx_ref.shape[2]
    x = x_ref[0]
    y = y_ref[0]
    z = z_ref[0]
    c = c_ref[0]
    dx = c[:, 0:1] - x
    dy = c[:, 1:2] - y
    dz = c[:, 2:3] - z
    d0 = (dx * dx + dy * dy) + dz * dz
    d_ref[...] = d0
    flane = lax.broadcasted_iota(jnp.int32, (GBLK, N), 1).astype(jnp.float32)
    mcol = lax.broadcasted_iota(jnp.int32, (GBLK, M), 1)
    bigf = jnp.float32(2.0 * N)
    mn0 = jnp.min(d0, axis=1, keepdims=True)

    def body(j4, carry):
        acc, mn = carry
        d = d_ref[...]
        for u in range(4):
            am = jnp.min(jnp.where(d == mn, flane, bigf), axis=1,
                         keepdims=True)
            d = jnp.where(flane == am, jnp.inf, d)
            mn = jnp.min(d, axis=1, keepdims=True)
            acc = acc + jnp.where(mcol == 4 * j4 + u,
                                  am.astype(jnp.int32), 0)
        d_ref[...] = d
        return acc, mn

    acc, _ = lax.fori_loop(
        0, M // 4, body, (jnp.zeros((GBLK, M), jnp.int32), mn0))
    idx_ref[0] = acc


def _knn_topk(x, y, z, center):
    B, N = x.shape
    G = NUM_GROUP_K
    M = GROUP_SIZE_K
    grid = (B, G // GBLK)
    x3 = x[:, None, :]
    y3 = y[:, None, :]
    z3 = z[:, None, :]
    return pl.pallas_call(
        _knn_body,
        grid=grid,
        in_specs=[
            pl.BlockSpec((1, 1, N), lambda b, g: (b, 0, 0)),
            pl.BlockSpec((1, 1, N), lambda b, g: (b, 0, 0)),
            pl.BlockSpec((1, 1, N), lambda b, g: (b, 0, 0)),
            pl.BlockSpec((1, GBLK, 3), lambda b, g: (b, g, 0)),
        ],
        out_specs=pl.BlockSpec((1, GBLK, M), lambda b, g: (b, g, 0)),
        out_shape=jax.ShapeDtypeStruct((B, G, M), jnp.int32),
        scratch_shapes=[pltpu.VMEM((GBLK, N), jnp.float32)],
    )(x3, y3, z3, center)


def _sc_gather_normalize(flat_idx, pts_pad, cent_pad):
    R = flat_idx.shape[0]
    info = plsc.get_sparse_core_info()
    nw = info.num_cores * info.num_subcores
    rpw = R // nw
    gpw = rpw // GROUP_SIZE_K
    mesh = plsc.VectorSubcoreMesh(core_axis_name="c", subcore_axis_name="s")

    @functools.partial(
        pl.kernel,
        mesh=mesh,
        compiler_params=pltpu.CompilerParams(use_tc_tiling_on_sc=False),
        out_type=jax.ShapeDtypeStruct((R, ROW_PAD), jnp.float32),
        scratch_types=[
            pltpu.VMEM((rpw,), jnp.int32),
            pltpu.VMEM((rpw, ROW_PAD), jnp.float32),
            pltpu.VMEM((gpw, ROW_PAD), jnp.float32),
            pltpu.SemaphoreType.DMA,
        ],
    )
    def k(idx_hbm, pts_hbm, cent_hbm, out_hbm, idx_v, rows_v, cent_v, sem):
        wid = lax.axis_index("s") * info.num_cores + lax.axis_index("c")
        rbase = wid * rpw
        pltpu.sync_copy(idx_hbm.at[pl.ds(rbase, rpw)], idx_v)
        pltpu.async_copy(pts_hbm.at[idx_v], rows_v, sem).wait()
        pltpu.sync_copy(cent_hbm.at[pl.ds(wid * gpw, gpw)], cent_v)

        def body(g, _):
            cvec = cent_v[g]
            base = g * GROUP_SIZE_K
            for j in range(GROUP_SIZE_K):
                rows_v[base + j] = rows_v[base + j] - cvec
            return 0

        lax.fori_loop(0, gpw, body, 0)
        pltpu.sync_copy(rows_v, out_hbm.at[pl.ds(rbase, rpw)])

    return k(flat_idx, pts_pad, cent_pad)


def kernel(pts):
    B, N, C = pts.shape
    G = NUM_GROUP_K
    M = GROUP_SIZE_K
    x = pts[:, :, 0]
    y = pts[:, :, 1]
    z = pts[:, :, 2]
    cx, cy, cz = _fps_centers(x, y, z)
    center = jnp.stack([cx, cy, cz], axis=-1)
    idx = _knn_topk(x, y, z, center)
    flat_idx = (idx + jnp.arange(B, dtype=jnp.int32)[:, None, None] * N
                ).reshape(B * G * M)
    pts_pad = jnp.pad(pts.reshape(B * N, C), ((0, 0), (0, ROW_PAD - C)))
    cent_pad = jnp.pad(center.reshape(B * G, 3), ((0, 0), (0, ROW_PAD - 3)))
    rows = _sc_gather_normalize(flat_idx, pts_pad, cent_pad)
    neighborhood = rows[:, :C].reshape(B, G, M, C)
    return neighborhood, center


# speedup vs baseline: 1.1161x; 1.0160x over previous
import functools

import jax
import jax.numpy as jnp
from jax import lax
from jax.experimental import pallas as pl
from jax.experimental.pallas import tpu as pltpu
from jax.experimental.pallas import tpu_sc as plsc

NUM_GROUP_K = 512
GROUP_SIZE_K = 32
ROW_PAD = 16
GBLK = 256


def _fps_body(x_ref, y_ref, z_ref, cx_ref, cy_ref, cz_ref, dist_ref):
    B, N = x_ref.shape
    G = cx_ref.shape[1]
    x = x_ref[...]
    y = y_ref[...]
    z = z_ref[...]
    flane = lax.broadcasted_iota(jnp.int32, (B, N), 1).astype(jnp.float32)
    gcol = lax.broadcasted_iota(jnp.int32, (B, G), 1)
    bigf = jnp.float32(2.0 * N)

    dist_ref[...] = jnp.full((B, N), jnp.inf, dtype=jnp.float32)
    lx0 = x[:, 0:1]
    ly0 = y[:, 0:1]
    lz0 = z[:, 0:1]
    cx0 = jnp.where(gcol == 0, lx0, 0.0)
    cy0 = jnp.where(gcol == 0, ly0, 0.0)
    cz0 = jnp.where(gcol == 0, lz0, 0.0)

    def step(j, carry):
        lx, ly, lz, cx, cy, cz = carry
        dx = x - lx
        dy = y - ly
        dz = z - lz
        d = (dx * dx + dy * dy) + dz * dz
        dist = jnp.minimum(dist_ref[...], d)
        dist_ref[...] = dist
        mx = jnp.max(dist, axis=1, keepdims=True)
        nxt = jnp.min(jnp.where(dist == mx, flane, bigf), axis=1, keepdims=True)
        sel = flane == nxt
        lx = jnp.sum(jnp.where(sel, x, 0.0), axis=1, keepdims=True)
        ly = jnp.sum(jnp.where(sel, y, 0.0), axis=1, keepdims=True)
        lz = jnp.sum(jnp.where(sel, z, 0.0), axis=1, keepdims=True)
        hit = gcol == j
        cx = cx + jnp.where(hit, lx, 0.0)
        cy = cy + jnp.where(hit, ly, 0.0)
        cz = cz + jnp.where(hit, lz, 0.0)
        return lx, ly, lz, cx, cy, cz

    _, _, _, cx, cy, cz = lax.fori_loop(
        1, G, step, (lx0, ly0, lz0, cx0, cy0, cz0))
    cx_ref[...] = cx
    cy_ref[...] = cy
    cz_ref[...] = cz


def _fps_centers(x, y, z):
    B, N = x.shape
    G = NUM_GROUP_K
    out = jax.ShapeDtypeStruct((B, G), jnp.float32)
    return pl.pallas_call(
        _fps_body,
        out_shape=(out, out, out),
        scratch_shapes=[pltpu.VMEM((B, N), jnp.float32)],
    )(x, y, z)


def _knn_body(x_ref, y_ref, z_ref, c_ref, idx_ref, d_ref):
    N = x_ref.shape[2]
    M = idx_ref.shape[2]
    x = x_ref[0]
    y = y_ref[0]
    z = z_ref[0]
    c = c_ref[0]
    dx = c[:, 0:1] - x
    dy = c[:, 1:2] - y
    dz = c[:, 2:3] - z
    d0 = (dx * dx + dy * dy) + dz * dz
    d_ref[...] = d0
    flane = lax.broadcasted_iota(jnp.int32, (GBLK, N), 1).astype(jnp.float32)
    mcol = lax.broadcasted_iota(jnp.int32, (GBLK, M), 1)
    bigf = jnp.float32(2.0 * N)
    mn0 = jnp.min(d0, axis=1, keepdims=True)

    def body(j4, carry):
        acc, mn = carry
        d = d_ref[...]
        for u in range(8):
            am = jnp.min(jnp.where(d == mn, flane, bigf), axis=1,
                         keepdims=True)
            d = jnp.where(flane == am, jnp.inf, d)
            mn = jnp.min(d, axis=1, keepdims=True)
            acc = acc + jnp.where(mcol == 8 * j4 + u,
                                  am.astype(jnp.int32), 0)
        d_ref[...] = d
        return acc, mn

    acc, _ = lax.fori_loop(
        0, M // 8, body, (jnp.zeros((GBLK, M), jnp.int32), mn0))
    idx_ref[0] = acc


def _knn_topk(x, y, z, center):
    B, N = x.shape
    G = NUM_GROUP_K
    M = GROUP_SIZE_K
    grid = (B, G // GBLK)
    x3 = x[:, None, :]
    y3 = y[:, None, :]
    z3 = z[:, None, :]
    return pl.pallas_call(
        _knn_body,
        grid=grid,
        in_specs=[
            pl.BlockSpec((1, 1, N), lambda b, g: (b, 0, 0)),
            pl.BlockSpec((1, 1, N), lambda b, g: (b, 0, 0)),
            pl.BlockSpec((1, 1, N), lambda b, g: (b, 0, 0)),
            pl.BlockSpec((1, GBLK, 3), lambda b, g: (b, g, 0)),
        ],
        out_specs=pl.BlockSpec((1, GBLK, M), lambda b, g: (b, g, 0)),
        out_shape=jax.ShapeDtypeStruct((B, G, M), jnp.int32),
        scratch_shapes=[pltpu.VMEM((GBLK, N), jnp.float32)],
    )(x3, y3, z3, center)


def _sc_gather_normalize(flat_idx, pts_pad, cent_pad):
    R = flat_idx.shape[0]
    info = plsc.get_sparse_core_info()
    nw = info.num_cores * info.num_subcores
    rpw = R // nw
    gpw = rpw // GROUP_SIZE_K
    mesh = plsc.VectorSubcoreMesh(core_axis_name="c", subcore_axis_name="s")

    @functools.partial(
        pl.kernel,
        mesh=mesh,
        compiler_params=pltpu.CompilerParams(use_tc_tiling_on_sc=False),
        out_type=jax.ShapeDtypeStruct((R, ROW_PAD), jnp.float32),
        scratch_types=[
            pltpu.VMEM((rpw,), jnp.int32),
            pltpu.VMEM((rpw, ROW_PAD), jnp.float32),
            pltpu.VMEM((gpw, ROW_PAD), jnp.float32),
            pltpu.SemaphoreType.DMA,
        ],
    )
    def k(idx_hbm, pts_hbm, cent_hbm, out_hbm, idx_v, rows_v, cent_v, sem):
        wid = lax.axis_index("s") * info.num_cores + lax.axis_index("c")
        rbase = wid * rpw
        pltpu.sync_copy(idx_hbm.at[pl.ds(rbase, rpw)], idx_v)
        pltpu.async_copy(pts_hbm.at[idx_v], rows_v, sem).wait()
        pltpu.sync_copy(cent_hbm.at[pl.ds(wid * gpw, gpw)], cent_v)

        def body(g, _):
            cvec = cent_v[g]
            base = g * GROUP_SIZE_K
            for j in range(GROUP_SIZE_K):
                rows_v[base + j] = rows_v[base + j] - cvec
            return 0

        lax.fori_loop(0, gpw, body, 0)
        pltpu.sync_copy(rows_v, out_hbm.at[pl.ds(rbase, rpw)])

    return k(flat_idx, pts_pad, cent_pad)


def kernel(pts):
    B, N, C = pts.shape
    G = NUM_GROUP_K
    M = GROUP_SIZE_K
    x = pts[:, :, 0]
    y = pts[:, :, 1]
    z = pts[:, :, 2]
    cx, cy, cz = _fps_centers(x, y, z)
    center = jnp.stack([cx, cy, cz], axis=-1)
    idx = _knn_topk(x, y, z, center)
    flat_idx = (idx + jnp.arange(B, dtype=jnp.int32)[:, None, None] * N
                ).reshape(B * G * M)
    pts_pad = jnp.pad(pts.reshape(B * N, C), ((0, 0), (0, ROW_PAD - C)))
    cent_pad = jnp.pad(center.reshape(B * G, 3), ((0, 0), (0, ROW_PAD - 3)))
    rows = _sc_gather_normalize(flat_idx, pts_pad, cent_pad)
    neighborhood = rows[:, :C].reshape(B, G, M, C)
    return neighborhood, center
